# baseline (speedup 1.0000x reference)
import functools

import jax
import jax.numpy as jnp
from jax import lax
from jax.experimental import pallas as pl
from jax.experimental.pallas import tpu as pltpu
from jax.experimental.pallas import tpu_sc as plsc

NC = 2
NS = 16
SCK = 128


def _mm_bias_body(x_ref, w_ref, b_ref, o_ref):
    o_ref[...] = (
        jnp.dot(x_ref[...], w_ref[...], preferred_element_type=jnp.float32)
        + b_ref[...]
    )


def _mm_bias(x, w, b2d, tile):
    n, d = x.shape
    return pl.pallas_call(
        _mm_bias_body,
        grid=(n // tile,),
        in_specs=[
            pl.BlockSpec((tile, d), lambda i: (i, 0)),
            pl.BlockSpec((d, w.shape[1]), lambda i: (0, 0)),
            pl.BlockSpec((1, w.shape[1]), lambda i: (0, 0)),
        ],
        out_specs=pl.BlockSpec((tile, w.shape[1]), lambda i: (i, 0)),
        out_shape=jax.ShapeDtypeStruct((n, w.shape[1]), jnp.float32),
    )(x, w, b2d)


def _node_update_body(s2_ref, c2_ref, x_ref, mw2_ref, mb2_ref, uw1a_ref,
                      uw1b_ref, ub1_ref, uw2_ref, ub2_ref, mw1a_ref,
                      x_out_ref, xa_out_ref):
    s = s2_ref[0] + s2_ref[1]
    cnt = c2_ref[0, :, 0] + c2_ref[1, :, 0]
    rinv = 1.0 / jnp.maximum(cnt, 1.0)
    agg = (
        jnp.dot(s, mw2_ref[...], preferred_element_type=jnp.float32)
        * rinv[:, None]
        + (cnt * rinv)[:, None] * mb2_ref[...]
    )
    h = jnp.maximum(
        jnp.dot(x_ref[...], uw1a_ref[...], preferred_element_type=jnp.float32)
        + jnp.dot(agg, uw1b_ref[...], preferred_element_type=jnp.float32)
        + ub1_ref[...],
        0.0,
    )
    x_new = (
        jnp.dot(h, uw2_ref[...], preferred_element_type=jnp.float32)
        + ub2_ref[...]
    )
    x_out_ref[...] = x_new
    xa_out_ref[...] = jnp.dot(
        x_new, mw1a_ref[...], preferred_element_type=jnp.float32
    )


def _node_update(s2, c2, x, mw2, mb2, uw1a, uw1b, ub1, uw2, ub2, mw1a, tile):
    n, d = x.shape
    full = lambda a: pl.BlockSpec(a.shape, lambda i: tuple(0 for _ in a.shape))
    return pl.pallas_call(
        _node_update_body,
        grid=(n // tile,),
        in_specs=[
            pl.BlockSpec((NC, tile, d), lambda i: (0, i, 0)),
            pl.BlockSpec((NC, tile, c2.shape[2]), lambda i: (0, i, 0)),
            pl.BlockSpec((tile, d), lambda i: (i, 0)),
            full(mw2), full(mb2), full(uw1a), full(uw1b), full(ub1),
            full(uw2), full(ub2), full(mw1a),
        ],
        out_specs=[
            pl.BlockSpec((tile, d), lambda i: (i, 0)),
            pl.BlockSpec((tile, d), lambda i: (i, 0)),
        ],
        out_shape=[
            jax.ShapeDtypeStruct((n, d), jnp.float32),
            jax.ShapeDtypeStruct((n, d), jnp.float32),
        ],
    )(s2, c2, x, mw2, mb2, uw1a, uw1b, ub1, uw2, ub2, mw1a)


def _edge_iter_body(nf, e_ref, g_ref, w1c_ref, w2_ref, b2_ref, o_ref):
    t = e_ref[...]
    g = g_ref[...]
    for _ in range(nf):
        h = jnp.maximum(
            jnp.dot(t, w1c_ref[...], preferred_element_type=jnp.float32) + g,
            0.0,
        )
        t = (
            jnp.dot(h, w2_ref[...], preferred_element_type=jnp.float32)
            + b2_ref[...]
        )
    o_ref[...] = t


def _edge_iter(e, gxr, w1c, w2, b2d, tile, nf):
    m, d = e.shape
    return pl.pallas_call(
        functools.partial(_edge_iter_body, nf),
        grid=(m // tile,),
        in_specs=[
            pl.BlockSpec((tile, d), lambda i: (i, 0)),
            pl.BlockSpec((tile, d), lambda i: (i, 0)),
            pl.BlockSpec((d, d), lambda i: (0, 0)),
            pl.BlockSpec((d, d), lambda i: (0, 0)),
            pl.BlockSpec((1, d), lambda i: (0, 0)),
        ],
        out_specs=pl.BlockSpec((tile, d), lambda i: (i, 0)),
        out_shape=jax.ShapeDtypeStruct((m, d), jnp.float32),
    )(e, gxr, w1c, w2, b2d)


def _pool_body(ng, x_ref, batch_ref, col_ref, e_ref, gc_ref, gw1a_ref,
               gw1b_ref, gw1c_ref, gb1_ref, gw2_ref, gb2_ref, o_ref,
               nsum_ref, ncnt_ref, esum_ref, ecnt_ref, gstart_ref):
    step = pl.program_id(0)
    nsteps = pl.num_programs(0)
    n = x_ref.shape[0]
    te = e_ref.shape[0]

    @pl.when(step == 0)
    def _():
        b = batch_ref[0, :]
        giota = lax.broadcasted_iota(jnp.int32, (ng, n), 0)
        onehot = (giota == b[None, :]).astype(jnp.float32)
        nsum_ref[...] = jnp.dot(
            onehot, x_ref[...], preferred_element_type=jnp.float32
        )
        ncnt_ref[...] = jnp.broadcast_to(
            jnp.sum(onehot, axis=1)[:, None], (ng, x_ref.shape[1])
        )
        esum_ref[...] = jnp.zeros_like(esum_ref)
        ecnt_ref[...] = jnp.zeros_like(ecnt_ref)
        for g in range(ng):
            gstart_ref[g] = jnp.sum((b < g).astype(jnp.int32))

    ct = col_ref[0, 0, :]
    bc = jnp.zeros((te,), jnp.int32)
    for g in range(1, ng):
        bc = bc + (ct >= gstart_ref[g]).astype(jnp.int32)
    m = (lax.broadcasted_iota(jnp.int32, (ng, te), 0) == bc[None, :]).astype(
        jnp.float32
    )
    esum_ref[...] += jnp.dot(
        m, e_ref[...], preferred_element_type=jnp.float32
    )
    ecnt_ref[...] += jnp.broadcast_to(
        jnp.sum(m, axis=1)[:, None], ecnt_ref.shape
    )

    @pl.when(step == nsteps - 1)
    def _():
        npool = nsum_ref[...] / jnp.maximum(ncnt_ref[...], 1.0)
        epool = esum_ref[...] / jnp.maximum(ecnt_ref[...], 1.0)
        h = jnp.maximum(
            jnp.dot(gc_ref[...], gw1a_ref[...],
                    preferred_element_type=jnp.float32)
            + jnp.dot(npool, gw1b_ref[...],
                      preferred_element_type=jnp.float32)
            + jnp.dot(epool, gw1c_ref[...],
                      preferred_element_type=jnp.float32)
            + gb1_ref[...],
            0.0,
        )
        o_ref[...] = (
            jnp.dot(h, gw2_ref[...], preferred_element_type=jnp.float32)
            + gb2_ref[...]
        )


def _pool_global(x, batch2d, col3, e, gc, gw1a, gw1b, gw1c, gb1, gw2, gb2,
                 tile):
    n, d = x.shape
    m = e.shape[0]
    ng = gc.shape[0]
    nb = m // tile
    full = lambda a: pl.BlockSpec(a.shape, lambda i: tuple(0 for _ in a.shape))
    return pl.pallas_call(
        functools.partial(_pool_body, ng),
        grid=(nb,),
        in_specs=[
            full(x),
            full(batch2d),
            pl.BlockSpec((1, 1, tile), lambda i: (i, 0, 0)),
            pl.BlockSpec((tile, d), lambda i: (i, 0)),
            full(gc), full(gw1a), full(gw1b), full(gw1c), full(gb1),
            full(gw2), full(gb2),
        ],
        out_specs=pl.BlockSpec((ng, d), lambda i: (0, 0)),
        out_shape=jax.ShapeDtypeStruct((ng, d), jnp.float32),
        scratch_shapes=[
            pltpu.VMEM((ng, d), jnp.float32),
            pltpu.VMEM((ng, d), jnp.float32),
            pltpu.VMEM((ng, d), jnp.float32),
            pltpu.VMEM((ng, d), jnp.float32),
            pltpu.SMEM((ng,), jnp.int32),
        ],
    )(x, batch2d, col3, e, gc, gw1a, gw1b, gw1c, gb1, gw2, gb2)


def _sc_mesh():
    return plsc.VectorSubcoreMesh(
        core_axis_name="c", subcore_axis_name="s",
        num_cores=NC, num_subcores=NS,
    )


def _make_node_msg(n, e, d):
    k = SCK
    n_chunks = e // k
    zchunk = 80
    n_zchunks = n // zchunk

    @functools.partial(
        pl.kernel,
        out_type=jax.ShapeDtypeStruct((NC, n, d), jnp.float32),
        mesh=_sc_mesh(),
        scratch_types=[
            pltpu.VMEM((k,), jnp.int32),
            pltpu.VMEM((k,), jnp.int32),
            pltpu.VMEM((k, d), jnp.float32),
            pltpu.VMEM((k, d), jnp.float32),
            pltpu.VMEM_SHARED((n, d), jnp.float32),
            pltpu.SemaphoreType.DMA,
            pltpu.SemaphoreType.DMA,
        ],
    )
    def node_msg(xa_hbm, ea_hbm, row_hbm, col_hbm, out_hbm,
                 idx_r, idx_c, gbuf, ebuf, acc, sem, sem2):
        c = lax.axis_index("c")
        s = lax.axis_index("s")
        wid = s * NC + c
        zeros16 = jnp.zeros((16,), jnp.float32)

        @pl.loop(0, k)
        def _(r):
            for j in range(d // 16):
                gbuf[r, pl.ds(j * 16, 16)] = zeros16

        @pl.loop(s, n_zchunks, step=NS)
        def _(zc):
            pltpu.sync_copy(
                gbuf.at[pl.ds(0, zchunk)],
                acc.at[pl.ds(zc * zchunk, zchunk)],
            )
        plsc.subcore_barrier()

        @pl.loop(wid, n_chunks, step=NC * NS)
        def _(chunk):
            base = chunk * k
            ea_cp = pltpu.async_copy(ea_hbm.at[pl.ds(base, k)], ebuf, sem2)
            pltpu.sync_copy(row_hbm.at[pl.ds(base, k)], idx_r)
            pltpu.sync_copy(col_hbm.at[pl.ds(base, k)], idx_c)
            g_cp = pltpu.async_copy(xa_hbm.at[idx_r], gbuf, sem)
            ea_cp.wait()
            g_cp.wait()

            @pl.loop(0, k, unroll=4)
            def _(r):
                for j in range(d // 16):
                    sl = pl.ds(j * 16, 16)
                    gbuf[r, sl] = jnp.maximum(gbuf[r, sl] + ebuf[r, sl], 0.0)

            pltpu.sync_copy(gbuf, acc.at[idx_c], add=True)

        plsc.subcore_barrier()

        @pl.loop(s, n_zchunks, step=NS)
        def _(zc):
            off = zc * zchunk
            pltpu.sync_copy(
                acc.at[pl.ds(off, zchunk)],
                out_hbm.at[c, pl.ds(off, zchunk)],
            )

    return node_msg


def _make_counts(n, e, w):
    k = SCK
    n_chunks = e // k
    zchunk = 80
    n_zchunks = n // zchunk

    @functools.partial(
        pl.kernel,
        out_type=jax.ShapeDtypeStruct((NC, n, w), jnp.float32),
        mesh=_sc_mesh(),
        scratch_types=[
            pltpu.VMEM((k,), jnp.int32),
            pltpu.VMEM((k, w), jnp.float32),
            pltpu.VMEM_SHARED((n, w), jnp.float32),
        ],
    )
    def counts(col_hbm, out_hbm, idx_c, obuf, acc):
        c = lax.axis_index("c")
        s = lax.axis_index("s")
        wid = s * NC + c
        zeros16 = jnp.zeros((16,), jnp.float32)

        @pl.loop(0, k)
        def _(r):
            for j in range(w // 16):
                obuf[r, pl.ds(j * 16, 16)] = zeros16

        @pl.loop(s, n_zchunks, step=NS)
        def _(zc):
            pltpu.sync_copy(
                obuf.at[pl.ds(0, zchunk)],
                acc.at[pl.ds(zc * zchunk, zchunk)],
            )
        plsc.subcore_barrier()

        ones16 = jnp.full((16,), 1.0, jnp.float32)

        @pl.loop(0, k)
        def _(r):
            for j in range(w // 16):
                obuf[r, pl.ds(j * 16, 16)] = ones16

        @pl.loop(wid, n_chunks, step=NC * NS)
        def _(chunk):
            pltpu.sync_copy(col_hbm.at[pl.ds(chunk * k, k)], idx_c)
            pltpu.sync_copy(obuf, acc.at[idx_c], add=True)

        plsc.subcore_barrier()

        @pl.loop(s, n_zchunks, step=NS)
        def _(zc):
            off = zc * zchunk
            pltpu.sync_copy(
                acc.at[pl.ds(off, zchunk)],
                out_hbm.at[c, pl.ds(off, zchunk)],
            )

    return counts


def _make_gxr(n, e, d):
    k = SCK
    n_chunks = e // k

    @functools.partial(
        pl.kernel,
        out_type=jax.ShapeDtypeStruct((e, d), jnp.float32),
        mesh=_sc_mesh(),
        scratch_types=[
            pltpu.VMEM((k,), jnp.int32),
            pltpu.VMEM((k,), jnp.int32),
            pltpu.VMEM((k, d), jnp.float32),
            pltpu.VMEM((k, d), jnp.float32),
            pltpu.SemaphoreType.DMA,
        ],
    )
    def gxr_kernel(xr_hbm, xc_hbm, row_hbm, col_hbm, out_hbm,
                   idx_r, idx_c, gbuf, ebuf, sem):
        c = lax.axis_index("c")
        s = lax.axis_index("s")
        wid = s * NC + c

        @pl.loop(wid, n_chunks, step=NC * NS)
        def _(chunk):
            base = chunk * k
            pltpu.sync_copy(row_hbm.at[pl.ds(base, k)], idx_r)
            pltpu.sync_copy(col_hbm.at[pl.ds(base, k)], idx_c)
            pltpu.async_copy(xr_hbm.at[idx_r], gbuf, sem).wait()
            pltpu.async_copy(xc_hbm.at[idx_c], ebuf, sem).wait()

            @pl.loop(0, k)
            def _(r):
                for j in range(d // 16):
                    sl = pl.ds(j * 16, 16)
                    gbuf[r, sl] = gbuf[r, sl] + ebuf[r, sl]

            pltpu.sync_copy(gbuf, out_hbm.at[pl.ds(base, k)])

    return gxr_kernel


NODE_MP_STEPS = 20
EDGE_MP_STEPS = 20


def kernel(x, edge_index, edge_attr, global_context, batch,
           msg_W1, msg_b1, msg_W2, msg_b2,
           upd_W1, upd_b1, upd_W2, upd_b2,
           edge_W1, edge_b1, edge_W2, edge_b2,
           glob_W1, glob_b1, glob_W2, glob_b2):
    n, d = x.shape
    e = edge_index.shape[1]
    row = edge_index[0]
    col = edge_index[1]

    r2 = lambda b: b.reshape(1, d)
    mW1a, mW1b = msg_W1[:d], msg_W1[d:]
    uW1a, uW1b = upd_W1[:d], upd_W1[d:]
    eW1a, eW1b, eW1c = edge_W1[:d], edge_W1[d:2 * d], edge_W1[2 * d:]
    gW1a, gW1b, gW1c = glob_W1[:d], glob_W1[d:2 * d], glob_W1[2 * d:]
    zero_b = jnp.zeros((1, d), jnp.float32)

    tile_n = 2000
    tile_e = 4000

    node_msg = _make_node_msg(n, e, d)
    counts = _make_counts(n, e, d)
    gxr_kernel = _make_gxr(n, e, d)


    ea = _mm_bias(edge_attr, mW1b, r2(msg_b1), tile_e)
    c2 = counts(col)

    xa = _mm_bias(x, mW1a, zero_b, tile_n)
    for _ in range(NODE_MP_STEPS):
        s2 = node_msg(xa, ea, row, col)
        x, xa = _node_update(
            s2, c2, x, msg_W2, r2(msg_b2), uW1a, uW1b, r2(upd_b1),
            upd_W2, r2(upd_b2), mW1a, tile_n,
        )

    xr = _mm_bias(x, eW1a, zero_b, tile_n)
    xc = _mm_bias(x, eW1b, r2(edge_b1), tile_n)
    gxr = gxr_kernel(xr, xc, row, col)
    ef = 4
    for _ in range(EDGE_MP_STEPS // ef):
        edge_attr = _edge_iter(edge_attr, gxr, eW1c, edge_W2, r2(edge_b2),
                               tile_e, ef)

    batch2d = batch.reshape(1, n)
    col3 = col.reshape(e // tile_e, 1, tile_e)
    gc = _pool_global(
        x, batch2d, col3, edge_attr, global_context,
        gW1a, gW1b, gW1c, r2(glob_b1), glob_W2, r2(glob_b2), tile_e,
    )
    return (x, edge_attr, gc)


# speedup vs baseline: 1.6634x; 1.6634x over previous
import functools

import jax
import jax.numpy as jnp
from jax import lax
from jax.experimental import pallas as pl
from jax.experimental.pallas import tpu as pltpu
from jax.experimental.pallas import tpu_sc as plsc

NC = 2
NS = 16
SCK = 128


def _mm_bias_body(x_ref, w_ref, b_ref, o_ref):
    o_ref[...] = (
        jnp.dot(x_ref[...], w_ref[...], preferred_element_type=jnp.float32)
        + b_ref[...]
    )


def _mm_bias(x, w, b2d, tile):
    n, d = x.shape
    return pl.pallas_call(
        _mm_bias_body,
        grid=(n // tile,),
        in_specs=[
            pl.BlockSpec((tile, d), lambda i: (i, 0)),
            pl.BlockSpec((d, w.shape[1]), lambda i: (0, 0)),
            pl.BlockSpec((1, w.shape[1]), lambda i: (0, 0)),
        ],
        out_specs=pl.BlockSpec((tile, w.shape[1]), lambda i: (i, 0)),
        out_shape=jax.ShapeDtypeStruct((n, w.shape[1]), jnp.float32),
    )(x, w, b2d)


def _node_update_body(s2_ref, c2_ref, x_ref, mw2_ref, mb2_ref, uw1a_ref,
                      uw1b_ref, ub1_ref, uw2_ref, ub2_ref, mw1a_ref,
                      x_out_ref, xa_out_ref):
    s = s2_ref[0] + s2_ref[1]
    cnt = c2_ref[0, :, 0] + c2_ref[1, :, 0]
    rinv = 1.0 / jnp.maximum(cnt, 1.0)
    agg = (
        jnp.dot(s, mw2_ref[...], preferred_element_type=jnp.float32)
        * rinv[:, None]
        + (cnt * rinv)[:, None] * mb2_ref[...]
    )
    h = jnp.maximum(
        jnp.dot(x_ref[...], uw1a_ref[...], preferred_element_type=jnp.float32)
        + jnp.dot(agg, uw1b_ref[...], preferred_element_type=jnp.float32)
        + ub1_ref[...],
        0.0,
    )
    x_new = (
        jnp.dot(h, uw2_ref[...], preferred_element_type=jnp.float32)
        + ub2_ref[...]
    )
    x_out_ref[...] = x_new
    xa_out_ref[...] = jnp.dot(
        x_new, mw1a_ref[...], preferred_element_type=jnp.float32
    )


def _node_update(s2, c2, x, mw2, mb2, uw1a, uw1b, ub1, uw2, ub2, mw1a, tile):
    n, d = x.shape
    full = lambda a: pl.BlockSpec(a.shape, lambda i: tuple(0 for _ in a.shape))
    return pl.pallas_call(
        _node_update_body,
        grid=(n // tile,),
        in_specs=[
            pl.BlockSpec((NC, tile, d), lambda i: (0, i, 0)),
            pl.BlockSpec((NC, tile, c2.shape[2]), lambda i: (0, i, 0)),
            pl.BlockSpec((tile, d), lambda i: (i, 0)),
            full(mw2), full(mb2), full(uw1a), full(uw1b), full(ub1),
            full(uw2), full(ub2), full(mw1a),
        ],
        out_specs=[
            pl.BlockSpec((tile, d), lambda i: (i, 0)),
            pl.BlockSpec((tile, d), lambda i: (i, 0)),
        ],
        out_shape=[
            jax.ShapeDtypeStruct((n, d), jnp.float32),
            jax.ShapeDtypeStruct((n, d), jnp.float32),
        ],
    )(s2, c2, x, mw2, mb2, uw1a, uw1b, ub1, uw2, ub2, mw1a)


def _edge_iter_body(nf, e_ref, g_ref, w1c_ref, w2_ref, b2_ref, o_ref):
    t = e_ref[...]
    g = g_ref[...]
    for _ in range(nf):
        h = jnp.maximum(
            jnp.dot(t, w1c_ref[...], preferred_element_type=jnp.float32) + g,
            0.0,
        )
        t = (
            jnp.dot(h, w2_ref[...], preferred_element_type=jnp.float32)
            + b2_ref[...]
        )
    o_ref[...] = t


def _edge_iter(e, gxr, w1c, w2, b2d, tile, nf):
    m, d = e.shape
    return pl.pallas_call(
        functools.partial(_edge_iter_body, nf),
        grid=(m // tile,),
        in_specs=[
            pl.BlockSpec((tile, d), lambda i: (i, 0)),
            pl.BlockSpec((tile, d), lambda i: (i, 0)),
            pl.BlockSpec((d, d), lambda i: (0, 0)),
            pl.BlockSpec((d, d), lambda i: (0, 0)),
            pl.BlockSpec((1, d), lambda i: (0, 0)),
        ],
        out_specs=pl.BlockSpec((tile, d), lambda i: (i, 0)),
        out_shape=jax.ShapeDtypeStruct((m, d), jnp.float32),
    )(e, gxr, w1c, w2, b2d)


def _pool_body(ng, x_ref, batch_ref, col_ref, e_ref, gc_ref, gw1a_ref,
               gw1b_ref, gw1c_ref, gb1_ref, gw2_ref, gb2_ref, o_ref,
               nsum_ref, ncnt_ref, esum_ref, ecnt_ref, gstart_ref):
    step = pl.program_id(0)
    nsteps = pl.num_programs(0)
    n = x_ref.shape[0]
    te = e_ref.shape[0]

    @pl.when(step == 0)
    def _():
        b = batch_ref[0, :]
        giota = lax.broadcasted_iota(jnp.int32, (ng, n), 0)
        onehot = (giota == b[None, :]).astype(jnp.float32)
        nsum_ref[...] = jnp.dot(
            onehot, x_ref[...], preferred_element_type=jnp.float32
        )
        ncnt_ref[...] = jnp.broadcast_to(
            jnp.sum(onehot, axis=1)[:, None], (ng, x_ref.shape[1])
        )
        esum_ref[...] = jnp.zeros_like(esum_ref)
        ecnt_ref[...] = jnp.zeros_like(ecnt_ref)
        for g in range(ng):
            gstart_ref[g] = jnp.sum((b < g).astype(jnp.int32))

    ct = col_ref[0, 0, :]
    bc = jnp.zeros((te,), jnp.int32)
    for g in range(1, ng):
        bc = bc + (ct >= gstart_ref[g]).astype(jnp.int32)
    m = (lax.broadcasted_iota(jnp.int32, (ng, te), 0) == bc[None, :]).astype(
        jnp.float32
    )
    esum_ref[...] += jnp.dot(
        m, e_ref[...], preferred_element_type=jnp.float32
    )
    ecnt_ref[...] += jnp.broadcast_to(
        jnp.sum(m, axis=1)[:, None], ecnt_ref.shape
    )

    @pl.when(step == nsteps - 1)
    def _():
        npool = nsum_ref[...] / jnp.maximum(ncnt_ref[...], 1.0)
        epool = esum_ref[...] / jnp.maximum(ecnt_ref[...], 1.0)
        h = jnp.maximum(
            jnp.dot(gc_ref[...], gw1a_ref[...],
                    preferred_element_type=jnp.float32)
            + jnp.dot(npool, gw1b_ref[...],
                      preferred_element_type=jnp.float32)
            + jnp.dot(epool, gw1c_ref[...],
                      preferred_element_type=jnp.float32)
            + gb1_ref[...],
            0.0,
        )
        o_ref[...] = (
            jnp.dot(h, gw2_ref[...], preferred_element_type=jnp.float32)
            + gb2_ref[...]
        )


def _pool_global(x, batch2d, col3, e, gc, gw1a, gw1b, gw1c, gb1, gw2, gb2,
                 tile):
    n, d = x.shape
    m = e.shape[0]
    ng = gc.shape[0]
    nb = m // tile
    full = lambda a: pl.BlockSpec(a.shape, lambda i: tuple(0 for _ in a.shape))
    return pl.pallas_call(
        functools.partial(_pool_body, ng),
        grid=(nb,),
        in_specs=[
            full(x),
            full(batch2d),
            pl.BlockSpec((1, 1, tile), lambda i: (i, 0, 0)),
            pl.BlockSpec((tile, d), lambda i: (i, 0)),
            full(gc), full(gw1a), full(gw1b), full(gw1c), full(gb1),
            full(gw2), full(gb2),
        ],
        out_specs=pl.BlockSpec((ng, d), lambda i: (0, 0)),
        out_shape=jax.ShapeDtypeStruct((ng, d), jnp.float32),
        scratch_shapes=[
            pltpu.VMEM((ng, d), jnp.float32),
            pltpu.VMEM((ng, d), jnp.float32),
            pltpu.VMEM((ng, d), jnp.float32),
            pltpu.VMEM((ng, d), jnp.float32),
            pltpu.SMEM((ng,), jnp.int32),
        ],
    )(x, batch2d, col3, e, gc, gw1a, gw1b, gw1c, gb1, gw2, gb2)


def _sc_mesh():
    return plsc.VectorSubcoreMesh(
        core_axis_name="c", subcore_axis_name="s",
        num_cores=NC, num_subcores=NS,
    )


def _make_node_msg(n, e, d):
    k = SCK
    n_chunks = e // k
    zchunk = 80
    n_zchunks = n // zchunk

    @functools.partial(
        pl.kernel,
        out_type=jax.ShapeDtypeStruct((NC, n, d), jnp.float32),
        mesh=_sc_mesh(),
        scratch_types=[
            pltpu.VMEM((k,), jnp.int32),
            pltpu.VMEM((k,), jnp.int32),
            pltpu.VMEM((k, d), jnp.float32),
            pltpu.VMEM((k, d), jnp.float32),
            pltpu.VMEM_SHARED((n, d), jnp.float32),
            pltpu.SemaphoreType.DMA,
            pltpu.SemaphoreType.DMA,
        ],
    )
    def node_msg(xa_hbm, ea_hbm, row_hbm, col_hbm, out_hbm,
                 idx_r, idx_c, gbuf, ebuf, acc, sem, sem2):
        c = lax.axis_index("c")
        s = lax.axis_index("s")
        wid = s * NC + c
        zeros16 = jnp.zeros((16,), jnp.float32)

        @pl.loop(0, k)
        def _(r):
            for j in range(d // 16):
                gbuf[r, pl.ds(j * 16, 16)] = zeros16

        @pl.loop(s, n_zchunks, step=NS)
        def _(zc):
            pltpu.sync_copy(
                gbuf.at[pl.ds(0, zchunk)],
                acc.at[pl.ds(zc * zchunk, zchunk)],
            )
        plsc.subcore_barrier()

        @pl.loop(wid, n_chunks, step=NC * NS)
        def _(chunk):
            base = chunk * k
            ea_cp = pltpu.async_copy(ea_hbm.at[pl.ds(base, k)], ebuf, sem2)
            pltpu.sync_copy(row_hbm.at[pl.ds(base, k)], idx_r)
            pltpu.sync_copy(col_hbm.at[pl.ds(base, k)], idx_c)
            g_cp = pltpu.async_copy(xa_hbm.at[idx_r], gbuf, sem)
            ea_cp.wait()
            g_cp.wait()

            @pl.loop(0, k)
            def _(r):
                for j in range(d // 16):
                    sl = pl.ds(j * 16, 16)
                    gbuf[r, sl] = jnp.maximum(gbuf[r, sl] + ebuf[r, sl], 0.0)

            pltpu.sync_copy(gbuf, acc.at[idx_c], add=True)

        plsc.subcore_barrier()

        @pl.loop(s, n_zchunks, step=NS)
        def _(zc):
            off = zc * zchunk
            pltpu.sync_copy(
                acc.at[pl.ds(off, zchunk)],
                out_hbm.at[c, pl.ds(off, zchunk)],
            )

    return node_msg


def _make_counts(n, e, w):
    k = SCK
    n_chunks = e // k
    zchunk = 80
    n_zchunks = n // zchunk

    @functools.partial(
        pl.kernel,
        out_type=jax.ShapeDtypeStruct((NC, n, w), jnp.float32),
        mesh=_sc_mesh(),
        scratch_types=[
            pltpu.VMEM((k,), jnp.int32),
            pltpu.VMEM((k, w), jnp.float32),
            pltpu.VMEM_SHARED((n, w), jnp.float32),
        ],
    )
    def counts(col_hbm, out_hbm, idx_c, obuf, acc):
        c = lax.axis_index("c")
        s = lax.axis_index("s")
        wid = s * NC + c
        zeros16 = jnp.zeros((16,), jnp.float32)

        @pl.loop(0, k)
        def _(r):
            for j in range(w // 16):
                obuf[r, pl.ds(j * 16, 16)] = zeros16

        @pl.loop(s, n_zchunks, step=NS)
        def _(zc):
            pltpu.sync_copy(
                obuf.at[pl.ds(0, zchunk)],
                acc.at[pl.ds(zc * zchunk, zchunk)],
            )
        plsc.subcore_barrier()

        ones16 = jnp.full((16,), 1.0, jnp.float32)

        @pl.loop(0, k)
        def _(r):
            for j in range(w // 16):
                obuf[r, pl.ds(j * 16, 16)] = ones16

        @pl.loop(wid, n_chunks, step=NC * NS)
        def _(chunk):
            pltpu.sync_copy(col_hbm.at[pl.ds(chunk * k, k)], idx_c)
            pltpu.sync_copy(obuf, acc.at[idx_c], add=True)

        plsc.subcore_barrier()

        @pl.loop(s, n_zchunks, step=NS)
        def _(zc):
            off = zc * zchunk
            pltpu.sync_copy(
                acc.at[pl.ds(off, zchunk)],
                out_hbm.at[c, pl.ds(off, zchunk)],
            )

    return counts


def _make_gxr(n, e, d):
    k = SCK
    n_chunks = e // k

    @functools.partial(
        pl.kernel,
        out_type=jax.ShapeDtypeStruct((e, d), jnp.float32),
        mesh=_sc_mesh(),
        scratch_types=[
            pltpu.VMEM((k,), jnp.int32),
            pltpu.VMEM((k,), jnp.int32),
            pltpu.VMEM((k, d), jnp.float32),
            pltpu.VMEM((k, d), jnp.float32),
            pltpu.SemaphoreType.DMA,
        ],
    )
    def gxr_kernel(xr_hbm, xc_hbm, row_hbm, col_hbm, out_hbm,
                   idx_r, idx_c, gbuf, ebuf, sem):
        c = lax.axis_index("c")
        s = lax.axis_index("s")
        wid = s * NC + c

        @pl.loop(wid, n_chunks, step=NC * NS)
        def _(chunk):
            base = chunk * k
            pltpu.sync_copy(row_hbm.at[pl.ds(base, k)], idx_r)
            pltpu.sync_copy(col_hbm.at[pl.ds(base, k)], idx_c)
            pltpu.async_copy(xr_hbm.at[idx_r], gbuf, sem).wait()
            pltpu.async_copy(xc_hbm.at[idx_c], ebuf, sem).wait()

            @pl.loop(0, k)
            def _(r):
                for j in range(d // 16):
                    sl = pl.ds(j * 16, 16)
                    gbuf[r, sl] = gbuf[r, sl] + ebuf[r, sl]

            pltpu.sync_copy(gbuf, out_hbm.at[pl.ds(base, k)])

    return gxr_kernel


NODE_MP_STEPS = 20
EDGE_MP_STEPS = 20


def kernel(x, edge_index, edge_attr, global_context, batch,
           msg_W1, msg_b1, msg_W2, msg_b2,
           upd_W1, upd_b1, upd_W2, upd_b2,
           edge_W1, edge_b1, edge_W2, edge_b2,
           glob_W1, glob_b1, glob_W2, glob_b2):
    n, d = x.shape
    e = edge_index.shape[1]
    row = edge_index[0]
    col = edge_index[1]

    r2 = lambda b: b.reshape(1, d)
    mW1a, mW1b = msg_W1[:d], msg_W1[d:]
    uW1a, uW1b = upd_W1[:d], upd_W1[d:]
    eW1a, eW1b, eW1c = edge_W1[:d], edge_W1[d:2 * d], edge_W1[2 * d:]
    gW1a, gW1b, gW1c = glob_W1[:d], glob_W1[d:2 * d], glob_W1[2 * d:]
    zero_b = jnp.zeros((1, d), jnp.float32)

    tile_n = 2000
    tile_e = 4000

    node_msg = _make_node_msg(n, e, d)
    counts = _make_counts(n, e, d)
    gxr_kernel = _make_gxr(n, e, d)


    ea = _mm_bias(edge_attr, mW1b, r2(msg_b1), tile_e)
    c2 = counts(col)

    xa = _mm_bias(x, mW1a, zero_b, tile_n)
    for _ in range(NODE_MP_STEPS):
        s2 = node_msg(xa, ea, row, col)
        x, xa = _node_update(
            s2, c2, x, msg_W2, r2(msg_b2), uW1a, uW1b, r2(upd_b1),
            upd_W2, r2(upd_b2), mW1a, tile_n,
        )

    xr = _mm_bias(x, eW1a, zero_b, tile_n)
    xc = _mm_bias(x, eW1b, r2(edge_b1), tile_n)
    gxr = gxr_kernel(xr, xc, row, col)
    ef = 4
    for _ in range(EDGE_MP_STEPS // ef):
        edge_attr = _edge_iter(edge_attr, gxr, eW1c, edge_W2, r2(edge_b2),
                               tile_e, ef)

    batch2d = batch.reshape(1, n)
    col3 = col.reshape(e // tile_e, 1, tile_e)
    gc = _pool_global(
        x, batch2d, col3, edge_attr, global_context,
        gW1a, gW1b, gW1c, r2(glob_b1), glob_W2, r2(glob_b2), tile_e,
    )
    return (x, edge_attr, gc)


# speedup vs baseline: 1.9403x; 1.1664x over previous
import functools

import jax
import jax.numpy as jnp
from jax import lax
from jax.experimental import pallas as pl
from jax.experimental.pallas import tpu as pltpu
from jax.experimental.pallas import tpu_sc as plsc

NC = 2
NS = 16
SCK = 128


def _mm_bias_body(x_ref, w_ref, b_ref, o_ref):
    o_ref[...] = (
        jnp.dot(x_ref[...], w_ref[...], preferred_element_type=jnp.float32)
        + b_ref[...]
    )


def _mm_bias(x, w, b2d, tile):
    n, d = x.shape
    return pl.pallas_call(
        _mm_bias_body,
        grid=(n // tile,),
        in_specs=[
            pl.BlockSpec((tile, d), lambda i: (i, 0)),
            pl.BlockSpec((d, w.shape[1]), lambda i: (0, 0)),
            pl.BlockSpec((1, w.shape[1]), lambda i: (0, 0)),
        ],
        out_specs=pl.BlockSpec((tile, w.shape[1]), lambda i: (i, 0)),
        out_shape=jax.ShapeDtypeStruct((n, w.shape[1]), jnp.float32),
    )(x, w, b2d)


def _node_update_body(s2_ref, c2_ref, x_ref, mw2_ref, mb2_ref, uw1a_ref,
                      uw1b_ref, ub1_ref, uw2_ref, ub2_ref, mw1a_ref,
                      x_out_ref, xa_out_ref):
    s = s2_ref[0] + s2_ref[1]
    cnt = c2_ref[0, :, 0] + c2_ref[1, :, 0]
    rinv = 1.0 / jnp.maximum(cnt, 1.0)
    agg = (
        jnp.dot(s, mw2_ref[...], preferred_element_type=jnp.float32)
        * rinv[:, None]
        + (cnt * rinv)[:, None] * mb2_ref[...]
    )
    h = jnp.maximum(
        jnp.dot(x_ref[...], uw1a_ref[...], preferred_element_type=jnp.float32)
        + jnp.dot(agg, uw1b_ref[...], preferred_element_type=jnp.float32)
        + ub1_ref[...],
        0.0,
    )
    x_new = (
        jnp.dot(h, uw2_ref[...], preferred_element_type=jnp.float32)
        + ub2_ref[...]
    )
    x_out_ref[...] = x_new
    xa_out_ref[...] = jnp.dot(
        x_new, mw1a_ref[...], preferred_element_type=jnp.float32
    )


def _node_update(s2, c2, x, mw2, mb2, uw1a, uw1b, ub1, uw2, ub2, mw1a, tile):
    n, d = x.shape
    full = lambda a: pl.BlockSpec(a.shape, lambda i: tuple(0 for _ in a.shape))
    return pl.pallas_call(
        _node_update_body,
        grid=(n // tile,),
        in_specs=[
            pl.BlockSpec((NC, tile, d), lambda i: (0, i, 0)),
            pl.BlockSpec((NC, tile, c2.shape[2]), lambda i: (0, i, 0)),
            pl.BlockSpec((tile, d), lambda i: (i, 0)),
            full(mw2), full(mb2), full(uw1a), full(uw1b), full(ub1),
            full(uw2), full(ub2), full(mw1a),
        ],
        out_specs=[
            pl.BlockSpec((tile, d), lambda i: (i, 0)),
            pl.BlockSpec((tile, d), lambda i: (i, 0)),
        ],
        out_shape=[
            jax.ShapeDtypeStruct((n, d), jnp.float32),
            jax.ShapeDtypeStruct((n, d), jnp.float32),
        ],
    )(s2, c2, x, mw2, mb2, uw1a, uw1b, ub1, uw2, ub2, mw1a)


def _edge_iter_body(nf, e_ref, g_ref, w1c_ref, w2_ref, b2_ref, o_ref):
    t = e_ref[...]
    g = g_ref[...]
    for _ in range(nf):
        h = jnp.maximum(
            jnp.dot(t, w1c_ref[...], preferred_element_type=jnp.float32) + g,
            0.0,
        )
        t = (
            jnp.dot(h, w2_ref[...], preferred_element_type=jnp.float32)
            + b2_ref[...]
        )
    o_ref[...] = t


def _edge_iter(e, gxr, w1c, w2, b2d, tile, nf):
    m, d = e.shape
    return pl.pallas_call(
        functools.partial(_edge_iter_body, nf),
        grid=(m // tile,),
        in_specs=[
            pl.BlockSpec((tile, d), lambda i: (i, 0)),
            pl.BlockSpec((tile, d), lambda i: (i, 0)),
            pl.BlockSpec((d, d), lambda i: (0, 0)),
            pl.BlockSpec((d, d), lambda i: (0, 0)),
            pl.BlockSpec((1, d), lambda i: (0, 0)),
        ],
        out_specs=pl.BlockSpec((tile, d), lambda i: (i, 0)),
        out_shape=jax.ShapeDtypeStruct((m, d), jnp.float32),
    )(e, gxr, w1c, w2, b2d)


def _pool_body(ng, x_ref, batch_ref, col_ref, e_ref, gc_ref, gw1a_ref,
               gw1b_ref, gw1c_ref, gb1_ref, gw2_ref, gb2_ref, o_ref,
               nsum_ref, ncnt_ref, esum_ref, ecnt_ref, gstart_ref):
    step = pl.program_id(0)
    nsteps = pl.num_programs(0)
    n = x_ref.shape[0]
    te = e_ref.shape[0]

    @pl.when(step == 0)
    def _():
        b = batch_ref[0, :]
        giota = lax.broadcasted_iota(jnp.int32, (ng, n), 0)
        onehot = (giota == b[None, :]).astype(jnp.float32)
        nsum_ref[...] = jnp.dot(
            onehot, x_ref[...], preferred_element_type=jnp.float32
        )
        ncnt_ref[...] = jnp.broadcast_to(
            jnp.sum(onehot, axis=1)[:, None], (ng, x_ref.shape[1])
        )
        esum_ref[...] = jnp.zeros_like(esum_ref)
        ecnt_ref[...] = jnp.zeros_like(ecnt_ref)
        for g in range(ng):
            gstart_ref[g] = jnp.sum((b < g).astype(jnp.int32))

    ct = col_ref[0, 0, :]
    bc = jnp.zeros((te,), jnp.int32)
    for g in range(1, ng):
        bc = bc + (ct >= gstart_ref[g]).astype(jnp.int32)
    m = (lax.broadcasted_iota(jnp.int32, (ng, te), 0) == bc[None, :]).astype(
        jnp.float32
    )
    esum_ref[...] += jnp.dot(
        m, e_ref[...], preferred_element_type=jnp.float32
    )
    ecnt_ref[...] += jnp.broadcast_to(
        jnp.sum(m, axis=1)[:, None], ecnt_ref.shape
    )

    @pl.when(step == nsteps - 1)
    def _():
        npool = nsum_ref[...] / jnp.maximum(ncnt_ref[...], 1.0)
        epool = esum_ref[...] / jnp.maximum(ecnt_ref[...], 1.0)
        h = jnp.maximum(
            jnp.dot(gc_ref[...], gw1a_ref[...],
                    preferred_element_type=jnp.float32)
            + jnp.dot(npool, gw1b_ref[...],
                      preferred_element_type=jnp.float32)
            + jnp.dot(epool, gw1c_ref[...],
                      preferred_element_type=jnp.float32)
            + gb1_ref[...],
            0.0,
        )
        o_ref[...] = (
            jnp.dot(h, gw2_ref[...], preferred_element_type=jnp.float32)
            + gb2_ref[...]
        )


def _pool_global(x, batch2d, col3, e, gc, gw1a, gw1b, gw1c, gb1, gw2, gb2,
                 tile):
    n, d = x.shape
    m = e.shape[0]
    ng = gc.shape[0]
    nb = m // tile
    full = lambda a: pl.BlockSpec(a.shape, lambda i: tuple(0 for _ in a.shape))
    return pl.pallas_call(
        functools.partial(_pool_body, ng),
        grid=(nb,),
        in_specs=[
            full(x),
            full(batch2d),
            pl.BlockSpec((1, 1, tile), lambda i: (i, 0, 0)),
            pl.BlockSpec((tile, d), lambda i: (i, 0)),
            full(gc), full(gw1a), full(gw1b), full(gw1c), full(gb1),
            full(gw2), full(gb2),
        ],
        out_specs=pl.BlockSpec((ng, d), lambda i: (0, 0)),
        out_shape=jax.ShapeDtypeStruct((ng, d), jnp.float32),
        scratch_shapes=[
            pltpu.VMEM((ng, d), jnp.float32),
            pltpu.VMEM((ng, d), jnp.float32),
            pltpu.VMEM((ng, d), jnp.float32),
            pltpu.VMEM((ng, d), jnp.float32),
            pltpu.SMEM((ng,), jnp.int32),
        ],
    )(x, batch2d, col3, e, gc, gw1a, gw1b, gw1c, gb1, gw2, gb2)


def _sc_mesh():
    return plsc.VectorSubcoreMesh(
        core_axis_name="c", subcore_axis_name="s",
        num_cores=NC, num_subcores=NS,
    )


def _make_node_msg(n, e, d):
    k = 80
    n_chunks = e // k
    zchunk = 80
    n_zchunks = n // zchunk

    nw = NC * NS
    max_iters = (n_chunks + nw - 1) // nw
    if max_iters % 2:
        max_iters += 1

    @functools.partial(
        pl.kernel,
        out_type=jax.ShapeDtypeStruct((NC, n, d), jnp.float32),
        mesh=_sc_mesh(),
        scratch_types=[
            pltpu.VMEM((k,), jnp.int32),
            pltpu.VMEM((k,), jnp.int32),
            pltpu.VMEM((k,), jnp.int32),
            pltpu.VMEM((k,), jnp.int32),
            pltpu.VMEM((k, d), jnp.float32),
            pltpu.VMEM((k, d), jnp.float32),
            pltpu.VMEM((k, d), jnp.float32),
            pltpu.VMEM((k, d), jnp.float32),
            pltpu.VMEM_SHARED((n, d), jnp.float32),
            pltpu.SemaphoreType.DMA,
            pltpu.SemaphoreType.DMA,
            pltpu.SemaphoreType.DMA,
            pltpu.SemaphoreType.DMA,
        ],
    )
    def node_msg(xa_hbm, ea_hbm, row_hbm, col_hbm, out_hbm,
                 idx_r0, idx_r1, idx_c0, idx_c1, gbuf0, gbuf1,
                 ebuf0, ebuf1, acc,
                 ea_sem0, ea_sem1, g_sem0, g_sem1):
        c = lax.axis_index("c")
        s = lax.axis_index("s")
        wid = s * NC + c
        zeros16 = jnp.zeros((16,), jnp.float32)
        idx_r = (idx_r0, idx_r1)
        idx_c = (idx_c0, idx_c1)
        gbuf = (gbuf0, gbuf1)
        ebuf = (ebuf0, ebuf1)
        ea_sem = (ea_sem0, ea_sem1)
        g_sem = (g_sem0, g_sem1)

        @pl.loop(0, k)
        def _(r):
            for j in range(d // 16):
                gbuf0[r, pl.ds(j * 16, 16)] = zeros16

        @pl.loop(s, n_zchunks, step=NS)
        def _(zc):
            pltpu.sync_copy(
                gbuf0.at[pl.ds(0, zchunk)],
                acc.at[pl.ds(zc * zchunk, zchunk)],
            )
        plsc.subcore_barrier()

        def issue(b, chunk):
            base = chunk * k
            pltpu.sync_copy(row_hbm.at[pl.ds(base, k)], idx_r[b])
            pltpu.sync_copy(col_hbm.at[pl.ds(base, k)], idx_c[b])
            pltpu.async_copy(ea_hbm.at[pl.ds(base, k)], ebuf[b], ea_sem[b])
            pltpu.async_copy(xa_hbm.at[idx_r[b]], gbuf[b], g_sem[b])

        for b in range(2):
            chunk = wid + nw * b

            @pl.when(chunk < n_chunks)
            def _():
                issue(b, chunk)

        @pl.loop(0, max_iters, step=2)
        def _(o):
            for b in range(2):
                i = o + b
                chunk = wid + nw * i

                @pl.when(chunk < n_chunks)
                def _():
                    base = chunk * k
                    pltpu.make_async_copy(
                        ea_hbm.at[pl.ds(base, k)], ebuf[b], ea_sem[b]
                    ).wait()
                    pltpu.make_async_copy(
                        xa_hbm.at[idx_r[b]], gbuf[b], g_sem[b]
                    ).wait()

                    @pl.loop(0, k)
                    def _(r):
                        for j in range(d // 16):
                            sl = pl.ds(j * 16, 16)
                            gbuf[b][r, sl] = jnp.maximum(
                                gbuf[b][r, sl] + ebuf[b][r, sl], 0.0
                            )

                    pltpu.sync_copy(gbuf[b], acc.at[idx_c[b]], add=True)

                @pl.when(chunk + 2 * nw < n_chunks)
                def _():
                    issue(b, chunk + 2 * nw)

        plsc.subcore_barrier()

        @pl.loop(s, n_zchunks, step=NS)
        def _(zc):
            off = zc * zchunk
            pltpu.sync_copy(
                acc.at[pl.ds(off, zchunk)],
                out_hbm.at[c, pl.ds(off, zchunk)],
            )

    return node_msg


def _make_counts(n, e, w):
    k = SCK
    n_chunks = e // k
    zchunk = 80
    n_zchunks = n // zchunk

    @functools.partial(
        pl.kernel,
        out_type=jax.ShapeDtypeStruct((NC, n, w), jnp.float32),
        mesh=_sc_mesh(),
        scratch_types=[
            pltpu.VMEM((k,), jnp.int32),
            pltpu.VMEM((k, w), jnp.float32),
            pltpu.VMEM_SHARED((n, w), jnp.float32),
        ],
    )
    def counts(col_hbm, out_hbm, idx_c, obuf, acc):
        c = lax.axis_index("c")
        s = lax.axis_index("s")
        wid = s * NC + c
        zeros16 = jnp.zeros((16,), jnp.float32)

        @pl.loop(0, k)
        def _(r):
            for j in range(w // 16):
                obuf[r, pl.ds(j * 16, 16)] = zeros16

        @pl.loop(s, n_zchunks, step=NS)
        def _(zc):
            pltpu.sync_copy(
                obuf.at[pl.ds(0, zchunk)],
                acc.at[pl.ds(zc * zchunk, zchunk)],
            )
        plsc.subcore_barrier()

        ones16 = jnp.full((16,), 1.0, jnp.float32)

        @pl.loop(0, k)
        def _(r):
            for j in range(w // 16):
                obuf[r, pl.ds(j * 16, 16)] = ones16

        @pl.loop(wid, n_chunks, step=NC * NS)
        def _(chunk):
            pltpu.sync_copy(col_hbm.at[pl.ds(chunk * k, k)], idx_c)
            pltpu.sync_copy(obuf, acc.at[idx_c], add=True)

        plsc.subcore_barrier()

        @pl.loop(s, n_zchunks, step=NS)
        def _(zc):
            off = zc * zchunk
            pltpu.sync_copy(
                acc.at[pl.ds(off, zchunk)],
                out_hbm.at[c, pl.ds(off, zchunk)],
            )

    return counts


def _make_gxr(n, e, d):
    k = SCK
    n_chunks = e // k

    @functools.partial(
        pl.kernel,
        out_type=jax.ShapeDtypeStruct((e, d), jnp.float32),
        mesh=_sc_mesh(),
        scratch_types=[
            pltpu.VMEM((k,), jnp.int32),
            pltpu.VMEM((k,), jnp.int32),
            pltpu.VMEM((k, d), jnp.float32),
            pltpu.VMEM((k, d), jnp.float32),
            pltpu.SemaphoreType.DMA,
        ],
    )
    def gxr_kernel(xr_hbm, xc_hbm, row_hbm, col_hbm, out_hbm,
                   idx_r, idx_c, gbuf, ebuf, sem):
        c = lax.axis_index("c")
        s = lax.axis_index("s")
        wid = s * NC + c

        @pl.loop(wid, n_chunks, step=NC * NS)
        def _(chunk):
            base = chunk * k
            pltpu.sync_copy(row_hbm.at[pl.ds(base, k)], idx_r)
            pltpu.sync_copy(col_hbm.at[pl.ds(base, k)], idx_c)
            pltpu.async_copy(xr_hbm.at[idx_r], gbuf, sem).wait()
            pltpu.async_copy(xc_hbm.at[idx_c], ebuf, sem).wait()

            @pl.loop(0, k)
            def _(r):
                for j in range(d // 16):
                    sl = pl.ds(j * 16, 16)
                    gbuf[r, sl] = gbuf[r, sl] + ebuf[r, sl]

            pltpu.sync_copy(gbuf, out_hbm.at[pl.ds(base, k)])

    return gxr_kernel


NODE_MP_STEPS = 20
EDGE_MP_STEPS = 20


def kernel(x, edge_index, edge_attr, global_context, batch,
           msg_W1, msg_b1, msg_W2, msg_b2,
           upd_W1, upd_b1, upd_W2, upd_b2,
           edge_W1, edge_b1, edge_W2, edge_b2,
           glob_W1, glob_b1, glob_W2, glob_b2):
    n, d = x.shape
    e = edge_index.shape[1]
    row = edge_index[0]
    col = edge_index[1]

    r2 = lambda b: b.reshape(1, d)
    mW1a, mW1b = msg_W1[:d], msg_W1[d:]
    uW1a, uW1b = upd_W1[:d], upd_W1[d:]
    eW1a, eW1b, eW1c = edge_W1[:d], edge_W1[d:2 * d], edge_W1[2 * d:]
    gW1a, gW1b, gW1c = glob_W1[:d], glob_W1[d:2 * d], glob_W1[2 * d:]
    zero_b = jnp.zeros((1, d), jnp.float32)

    tile_n = 2000
    tile_e = 4000

    node_msg = _make_node_msg(n, e, d)
    counts = _make_counts(n, e, d)
    gxr_kernel = _make_gxr(n, e, d)


    ea = _mm_bias(edge_attr, mW1b, r2(msg_b1), tile_e)
    c2 = counts(col)

    xa = _mm_bias(x, mW1a, zero_b, tile_n)
    for _ in range(NODE_MP_STEPS):
        s2 = node_msg(xa, ea, row, col)
        x, xa = _node_update(
            s2, c2, x, msg_W2, r2(msg_b2), uW1a, uW1b, r2(upd_b1),
            upd_W2, r2(upd_b2), mW1a, tile_n,
        )

    xr = _mm_bias(x, eW1a, zero_b, tile_n)
    xc = _mm_bias(x, eW1b, r2(edge_b1), tile_n)
    gxr = gxr_kernel(xr, xc, row, col)
    ef = 4
    for _ in range(EDGE_MP_STEPS // ef):
        edge_attr = _edge_iter(edge_attr, gxr, eW1c, edge_W2, r2(edge_b2),
                               tile_e, ef)

    batch2d = batch.reshape(1, n)
    col3 = col.reshape(e // tile_e, 1, tile_e)
    gc = _pool_global(
        x, batch2d, col3, edge_attr, global_context,
        gW1a, gW1b, gW1c, r2(glob_b1), glob_W2, r2(glob_b2), tile_e,
    )
    return (x, edge_attr, gc)


# speedup vs baseline: 2.4271x; 1.2509x over previous
import functools

import jax
import jax.numpy as jnp
from jax import lax
from jax.experimental import pallas as pl
from jax.experimental.pallas import tpu as pltpu
from jax.experimental.pallas import tpu_sc as plsc

NC = 2
NS = 16
SCK = 128


def _mm_bias_body(x_ref, w_ref, b_ref, o_ref):
    o_ref[...] = (
        jnp.dot(x_ref[...], w_ref[...], preferred_element_type=jnp.float32)
        + b_ref[...]
    )


def _mm_bias(x, w, b2d, tile):
    n, d = x.shape
    return pl.pallas_call(
        _mm_bias_body,
        grid=(n // tile,),
        in_specs=[
            pl.BlockSpec((tile, d), lambda i: (i, 0)),
            pl.BlockSpec((d, w.shape[1]), lambda i: (0, 0)),
            pl.BlockSpec((1, w.shape[1]), lambda i: (0, 0)),
        ],
        out_specs=pl.BlockSpec((tile, w.shape[1]), lambda i: (i, 0)),
        out_shape=jax.ShapeDtypeStruct((n, w.shape[1]), jnp.float32),
    )(x, w, b2d)


def _node_update_body(s2_ref, c2_ref, x_ref, mw2_ref, mb2_ref, uw1a_ref,
                      uw1b_ref, ub1_ref, uw2_ref, ub2_ref, mw1a_ref,
                      x_out_ref, xa_out_ref):
    s = s2_ref[0] + s2_ref[1]
    cnt = c2_ref[0, :, 0] + c2_ref[1, :, 0]
    rinv = 1.0 / jnp.maximum(cnt, 1.0)
    agg = (
        jnp.dot(s, mw2_ref[...], preferred_element_type=jnp.float32)
        * rinv[:, None]
        + (cnt * rinv)[:, None] * mb2_ref[...]
    )
    h = jnp.maximum(
        jnp.dot(x_ref[...], uw1a_ref[...], preferred_element_type=jnp.float32)
        + jnp.dot(agg, uw1b_ref[...], preferred_element_type=jnp.float32)
        + ub1_ref[...],
        0.0,
    )
    x_new = (
        jnp.dot(h, uw2_ref[...], preferred_element_type=jnp.float32)
        + ub2_ref[...]
    )
    x_out_ref[...] = x_new
    xa_out_ref[...] = jnp.dot(
        x_new, mw1a_ref[...], preferred_element_type=jnp.float32
    )


def _node_update(s2, c2, x, mw2, mb2, uw1a, uw1b, ub1, uw2, ub2, mw1a, tile):
    n, d = x.shape
    full = lambda a: pl.BlockSpec(a.shape, lambda i: tuple(0 for _ in a.shape))
    return pl.pallas_call(
        _node_update_body,
        grid=(n // tile,),
        in_specs=[
            pl.BlockSpec((NC, tile, d), lambda i: (0, i, 0)),
            pl.BlockSpec((NC, tile, c2.shape[2]), lambda i: (0, i, 0)),
            pl.BlockSpec((tile, d), lambda i: (i, 0)),
            full(mw2), full(mb2), full(uw1a), full(uw1b), full(ub1),
            full(uw2), full(ub2), full(mw1a),
        ],
        out_specs=[
            pl.BlockSpec((tile, d), lambda i: (i, 0)),
            pl.BlockSpec((tile, d), lambda i: (i, 0)),
        ],
        out_shape=[
            jax.ShapeDtypeStruct((n, d), jnp.float32),
            jax.ShapeDtypeStruct((n, d), jnp.float32),
        ],
    )(s2, c2, x, mw2, mb2, uw1a, uw1b, ub1, uw2, ub2, mw1a)


def _edge_iter_body(nf, e_ref, g_ref, w1c_ref, w2_ref, b2_ref, o_ref):
    t = e_ref[...]
    g = g_ref[...]
    for _ in range(nf):
        h = jnp.maximum(
            jnp.dot(t, w1c_ref[...], preferred_element_type=jnp.float32) + g,
            0.0,
        )
        t = (
            jnp.dot(h, w2_ref[...], preferred_element_type=jnp.float32)
            + b2_ref[...]
        )
    o_ref[...] = t


def _edge_iter(e, gxr, w1c, w2, b2d, tile, nf):
    m, d = e.shape
    return pl.pallas_call(
        functools.partial(_edge_iter_body, nf),
        grid=(m // tile,),
        in_specs=[
            pl.BlockSpec((tile, d), lambda i: (i, 0)),
            pl.BlockSpec((tile, d), lambda i: (i, 0)),
            pl.BlockSpec((d, d), lambda i: (0, 0)),
            pl.BlockSpec((d, d), lambda i: (0, 0)),
            pl.BlockSpec((1, d), lambda i: (0, 0)),
        ],
        out_specs=pl.BlockSpec((tile, d), lambda i: (i, 0)),
        out_shape=jax.ShapeDtypeStruct((m, d), jnp.float32),
    )(e, gxr, w1c, w2, b2d)


def _pool_body(ng, x_ref, batch_ref, col_ref, e_ref, gc_ref, gw1a_ref,
               gw1b_ref, gw1c_ref, gb1_ref, gw2_ref, gb2_ref, o_ref,
               nsum_ref, ncnt_ref, esum_ref, ecnt_ref, gstart_ref):
    step = pl.program_id(0)
    nsteps = pl.num_programs(0)
    n = x_ref.shape[0]
    te = e_ref.shape[0]

    @pl.when(step == 0)
    def _():
        b = batch_ref[0, :]
        giota = lax.broadcasted_iota(jnp.int32, (ng, n), 0)
        onehot = (giota == b[None, :]).astype(jnp.float32)
        nsum_ref[...] = jnp.dot(
            onehot, x_ref[...], preferred_element_type=jnp.float32
        )
        ncnt_ref[...] = jnp.broadcast_to(
            jnp.sum(onehot, axis=1)[:, None], (ng, x_ref.shape[1])
        )
        esum_ref[...] = jnp.zeros_like(esum_ref)
        ecnt_ref[...] = jnp.zeros_like(ecnt_ref)
        for g in range(ng):
            gstart_ref[g] = jnp.sum((b < g).astype(jnp.int32))

    ct = col_ref[0, 0, :]
    bc = jnp.zeros((te,), jnp.int32)
    for g in range(1, ng):
        bc = bc + (ct >= gstart_ref[g]).astype(jnp.int32)
    m = (lax.broadcasted_iota(jnp.int32, (ng, te), 0) == bc[None, :]).astype(
        jnp.float32
    )
    esum_ref[...] += jnp.dot(
        m, e_ref[...], preferred_element_type=jnp.float32
    )
    ecnt_ref[...] += jnp.broadcast_to(
        jnp.sum(m, axis=1)[:, None], ecnt_ref.shape
    )

    @pl.when(step == nsteps - 1)
    def _():
        npool = nsum_ref[...] / jnp.maximum(ncnt_ref[...], 1.0)
        epool = esum_ref[...] / jnp.maximum(ecnt_ref[...], 1.0)
        h = jnp.maximum(
            jnp.dot(gc_ref[...], gw1a_ref[...],
                    preferred_element_type=jnp.float32)
            + jnp.dot(npool, gw1b_ref[...],
                      preferred_element_type=jnp.float32)
            + jnp.dot(epool, gw1c_ref[...],
                      preferred_element_type=jnp.float32)
            + gb1_ref[...],
            0.0,
        )
        o_ref[...] = (
            jnp.dot(h, gw2_ref[...], preferred_element_type=jnp.float32)
            + gb2_ref[...]
        )


def _pool_global(x, batch2d, col3, e, gc, gw1a, gw1b, gw1c, gb1, gw2, gb2,
                 tile):
    n, d = x.shape
    m = e.shape[0]
    ng = gc.shape[0]
    nb = m // tile
    full = lambda a: pl.BlockSpec(a.shape, lambda i: tuple(0 for _ in a.shape))
    return pl.pallas_call(
        functools.partial(_pool_body, ng),
        grid=(nb,),
        in_specs=[
            full(x),
            full(batch2d),
            pl.BlockSpec((1, 1, tile), lambda i: (i, 0, 0)),
            pl.BlockSpec((tile, d), lambda i: (i, 0)),
            full(gc), full(gw1a), full(gw1b), full(gw1c), full(gb1),
            full(gw2), full(gb2),
        ],
        out_specs=pl.BlockSpec((ng, d), lambda i: (0, 0)),
        out_shape=jax.ShapeDtypeStruct((ng, d), jnp.float32),
        scratch_shapes=[
            pltpu.VMEM((ng, d), jnp.float32),
            pltpu.VMEM((ng, d), jnp.float32),
            pltpu.VMEM((ng, d), jnp.float32),
            pltpu.VMEM((ng, d), jnp.float32),
            pltpu.SMEM((ng,), jnp.int32),
        ],
    )(x, batch2d, col3, e, gc, gw1a, gw1b, gw1c, gb1, gw2, gb2)


def _sc_mesh():
    return plsc.VectorSubcoreMesh(
        core_axis_name="c", subcore_axis_name="s",
        num_cores=NC, num_subcores=NS,
    )


def _make_node_msg(n, e, d):
    k = 80
    n_chunks = e // k
    zchunk = 80
    n_zchunks = n // zchunk

    nw = NC * NS
    max_iters = (n_chunks + nw - 1) // nw
    max_iters += (-max_iters) % 4

    @functools.partial(
        pl.kernel,
        out_type=jax.ShapeDtypeStruct((NC, n, d), jnp.float32),
        mesh=_sc_mesh(),
        scratch_types=(
            [pltpu.VMEM((k,), jnp.int32)] * 8
            + [pltpu.VMEM((k, d), jnp.float32)] * 4
            + [pltpu.VMEM_SHARED((n, d), jnp.float32)]
            + [pltpu.SemaphoreType.DMA] * 8
        ),
    )
    def node_msg(xa_hbm, ea_hbm, row_hbm, col_hbm, out_hbm,
                 ir0, ir1, ir2, ir3, ic0, ic1, ic2, ic3,
                 gbuf0, gbuf1, ebuf0, ebuf1, acc,
                 eas0, eas1, gs0, gs1, is0, is1, is2, is3):
        c = lax.axis_index("c")
        s = lax.axis_index("s")
        wid = s * NC + c
        zeros16 = jnp.zeros((16,), jnp.float32)
        idx_r = (ir0, ir1, ir2, ir3)
        idx_c = (ic0, ic1, ic2, ic3)
        gbuf = (gbuf0, gbuf1)
        ebuf = (ebuf0, ebuf1)
        ea_sem = (eas0, eas1)
        g_sem = (gs0, gs1)
        idx_sem = (is0, is1, is2, is3)

        @pl.loop(0, k)
        def _(r):
            for j in range(d // 16):
                gbuf0[r, pl.ds(j * 16, 16)] = zeros16

        @pl.loop(s, n_zchunks, step=NS)
        def _(zc):
            pltpu.sync_copy(
                gbuf0.at[pl.ds(0, zchunk)],
                acc.at[pl.ds(zc * zchunk, zchunk)],
            )
        plsc.subcore_barrier()

        def issue_idx(q, chunk):
            base = chunk * k
            pltpu.async_copy(row_hbm.at[pl.ds(base, k)], idx_r[q], idx_sem[q])
            pltpu.async_copy(col_hbm.at[pl.ds(base, k)], idx_c[q], idx_sem[q])

        def wait_idx(q, chunk):
            base = chunk * k
            pltpu.make_async_copy(
                row_hbm.at[pl.ds(base, k)], idx_r[q], idx_sem[q]
            ).wait()
            pltpu.make_async_copy(
                col_hbm.at[pl.ds(base, k)], idx_c[q], idx_sem[q]
            ).wait()

        def issue_data(b, q, chunk):
            base = chunk * k
            pltpu.async_copy(ea_hbm.at[pl.ds(base, k)], ebuf[b], ea_sem[b])
            pltpu.async_copy(xa_hbm.at[idx_r[q]], gbuf[b], g_sem[b])

        for i in range(3):
            chunk = wid + nw * i

            @pl.when(chunk < n_chunks)
            def _(i=i, chunk=chunk):
                issue_idx(i % 4, chunk)

        for i in range(2):
            chunk = wid + nw * i

            @pl.when(chunk < n_chunks)
            def _(i=i, chunk=chunk):
                wait_idx(i % 4, chunk)
                issue_data(i % 2, i % 4, chunk)

        @pl.loop(0, max_iters, step=4)
        def _(o):
            for q in range(4):
                b = q % 2
                i = o + q
                chunk = wid + nw * i

                @pl.when(chunk < n_chunks)
                def _(b=b, q=q, chunk=chunk):
                    base = chunk * k
                    pltpu.make_async_copy(
                        ea_hbm.at[pl.ds(base, k)], ebuf[b], ea_sem[b]
                    ).wait()
                    pltpu.make_async_copy(
                        xa_hbm.at[idx_r[q]], gbuf[b], g_sem[b]
                    ).wait()

                    @pl.loop(0, k)
                    def _(r):
                        for j in range(d // 16):
                            sl = pl.ds(j * 16, 16)
                            gbuf[b][r, sl] = jnp.maximum(
                                gbuf[b][r, sl] + ebuf[b][r, sl], 0.0
                            )

                    pltpu.sync_copy(gbuf[b], acc.at[idx_c[q]], add=True)

                @pl.when(chunk + 3 * nw < n_chunks)
                def _(q=q, chunk=chunk):
                    issue_idx((q + 3) % 4, chunk + 3 * nw)

                @pl.when(chunk + 2 * nw < n_chunks)
                def _(b=b, q=q, chunk=chunk):
                    wait_idx((q + 2) % 4, chunk + 2 * nw)
                    issue_data(b, (q + 2) % 4, chunk + 2 * nw)

        plsc.subcore_barrier()

        @pl.loop(s, n_zchunks, step=NS)
        def _(zc):
            off = zc * zchunk
            pltpu.sync_copy(
                acc.at[pl.ds(off, zchunk)],
                out_hbm.at[c, pl.ds(off, zchunk)],
            )

    return node_msg


def _make_counts(n, e, w):
    k = SCK
    n_chunks = e // k
    zchunk = 80
    n_zchunks = n // zchunk

    @functools.partial(
        pl.kernel,
        out_type=jax.ShapeDtypeStruct((NC, n, w), jnp.float32),
        mesh=_sc_mesh(),
        scratch_types=[
            pltpu.VMEM((k,), jnp.int32),
            pltpu.VMEM((k, w), jnp.float32),
            pltpu.VMEM_SHARED((n, w), jnp.float32),
        ],
    )
    def counts(col_hbm, out_hbm, idx_c, obuf, acc):
        c = lax.axis_index("c")
        s = lax.axis_index("s")
        wid = s * NC + c
        zeros16 = jnp.zeros((16,), jnp.float32)

        @pl.loop(0, k)
        def _(r):
            for j in range(w // 16):
                obuf[r, pl.ds(j * 16, 16)] = zeros16

        @pl.loop(s, n_zchunks, step=NS)
        def _(zc):
            pltpu.sync_copy(
                obuf.at[pl.ds(0, zchunk)],
                acc.at[pl.ds(zc * zchunk, zchunk)],
            )
        plsc.subcore_barrier()

        ones16 = jnp.full((16,), 1.0, jnp.float32)

        @pl.loop(0, k)
        def _(r):
            for j in range(w // 16):
                obuf[r, pl.ds(j * 16, 16)] = ones16

        @pl.loop(wid, n_chunks, step=NC * NS)
        def _(chunk):
            pltpu.sync_copy(col_hbm.at[pl.ds(chunk * k, k)], idx_c)
            pltpu.sync_copy(obuf, acc.at[idx_c], add=True)

        plsc.subcore_barrier()

        @pl.loop(s, n_zchunks, step=NS)
        def _(zc):
            off = zc * zchunk
            pltpu.sync_copy(
                acc.at[pl.ds(off, zchunk)],
                out_hbm.at[c, pl.ds(off, zchunk)],
            )

    return counts


def _make_gxr(n, e, d):
    k = SCK
    n_chunks = e // k

    @functools.partial(
        pl.kernel,
        out_type=jax.ShapeDtypeStruct((e, d), jnp.float32),
        mesh=_sc_mesh(),
        scratch_types=[
            pltpu.VMEM((k,), jnp.int32),
            pltpu.VMEM((k,), jnp.int32),
            pltpu.VMEM((k, d), jnp.float32),
            pltpu.VMEM((k, d), jnp.float32),
            pltpu.SemaphoreType.DMA,
        ],
    )
    def gxr_kernel(xr_hbm, xc_hbm, row_hbm, col_hbm, out_hbm,
                   idx_r, idx_c, gbuf, ebuf, sem):
        c = lax.axis_index("c")
        s = lax.axis_index("s")
        wid = s * NC + c

        @pl.loop(wid, n_chunks, step=NC * NS)
        def _(chunk):
            base = chunk * k
            pltpu.sync_copy(row_hbm.at[pl.ds(base, k)], idx_r)
            pltpu.sync_copy(col_hbm.at[pl.ds(base, k)], idx_c)
            pltpu.async_copy(xr_hbm.at[idx_r], gbuf, sem).wait()
            pltpu.async_copy(xc_hbm.at[idx_c], ebuf, sem).wait()

            @pl.loop(0, k)
            def _(r):
                for j in range(d // 16):
                    sl = pl.ds(j * 16, 16)
                    gbuf[r, sl] = gbuf[r, sl] + ebuf[r, sl]

            pltpu.sync_copy(gbuf, out_hbm.at[pl.ds(base, k)])

    return gxr_kernel


NODE_MP_STEPS = 20
EDGE_MP_STEPS = 20


def kernel(x, edge_index, edge_attr, global_context, batch,
           msg_W1, msg_b1, msg_W2, msg_b2,
           upd_W1, upd_b1, upd_W2, upd_b2,
           edge_W1, edge_b1, edge_W2, edge_b2,
           glob_W1, glob_b1, glob_W2, glob_b2):
    n, d = x.shape
    e = edge_index.shape[1]
    row = edge_index[0]
    col = edge_index[1]

    r2 = lambda b: b.reshape(1, d)
    mW1a, mW1b = msg_W1[:d], msg_W1[d:]
    uW1a, uW1b = upd_W1[:d], upd_W1[d:]
    eW1a, eW1b, eW1c = edge_W1[:d], edge_W1[d:2 * d], edge_W1[2 * d:]
    gW1a, gW1b, gW1c = glob_W1[:d], glob_W1[d:2 * d], glob_W1[2 * d:]
    zero_b = jnp.zeros((1, d), jnp.float32)

    tile_n = 2000
    tile_e = 4000

    node_msg = _make_node_msg(n, e, d)
    counts = _make_counts(n, e, d)
    gxr_kernel = _make_gxr(n, e, d)


    ea = _mm_bias(edge_attr, mW1b, r2(msg_b1), tile_e)
    c2 = counts(col)

    xa = _mm_bias(x, mW1a, zero_b, tile_n)
    for _ in range(NODE_MP_STEPS):
        s2 = node_msg(xa, ea, row, col)
        x, xa = _node_update(
            s2, c2, x, msg_W2, r2(msg_b2), uW1a, uW1b, r2(upd_b1),
            upd_W2, r2(upd_b2), mW1a, tile_n,
        )

    xr = _mm_bias(x, eW1a, zero_b, tile_n)
    xc = _mm_bias(x, eW1b, r2(edge_b1), tile_n)
    gxr = gxr_kernel(xr, xc, row, col)
    ef = 4
    for _ in range(EDGE_MP_STEPS // ef):
        edge_attr = _edge_iter(edge_attr, gxr, eW1c, edge_W2, r2(edge_b2),
                               tile_e, ef)

    batch2d = batch.reshape(1, n)
    col3 = col.reshape(e // tile_e, 1, tile_e)
    gc = _pool_global(
        x, batch2d, col3, edge_attr, global_context,
        gW1a, gW1b, gW1c, r2(glob_b1), glob_W2, r2(glob_b2), tile_e,
    )
    return (x, edge_attr, gc)
